# baseline (speedup 1.0000x reference)
import functools

import jax
import jax.numpy as jnp
from jax import lax
from jax.experimental import pallas as pl
from jax.experimental.pallas import tpu as pltpu
from jax.experimental.pallas import tpu_sc as plsc

NUM_HEADS = 16
SEQ = 2048
TBL = 1023
TBL_FLAT = TBL * NUM_HEADS
EXT_PITCH = 4352
NUM_SHIFTS = 8
LANES = 16
PAL_H = 8
PAL_W = 4088
ROWS_PER_WORKER = PAL_H // 2
FIRE = 4
CHUNKS = ROWS_PER_WORKER // FIRE


def _sc_body(table_hbm, pal_hbm, tbl_v, ext_v, sem):
    head = lax.axis_index("s")
    half = lax.axis_index("c")

    pltpu.sync_copy(table_hbm, tbl_v.at[pl.ds(0, TBL_FLAT)])

    lane = lax.iota(jnp.int32, LANES)
    row_base = half * ROWS_PER_WORKER
    q0 = (PAL_H - 1) - row_base

    def build(it, _):
        base = it * (2 * LANES)
        for u in range(2):
            off = base + u * LANES
            pos = off + lane
            for j in range(ROWS_PER_WORKER):
                r_idx = jnp.clip(pos + (q0 - j) + (-1536), 0, TBL - 1)
                vals = plsc.load_gather(tbl_v, [r_idx * NUM_HEADS + head])
                ext_v[j, pl.ds(off, LANES)] = vals
        return 0

    lax.fori_loop(0, EXT_PITCH // (2 * LANES), build, 0)

    pltpu.async_copy(
        ext_v.at[:, pl.ds(0, PAL_W)],
        pal_hbm.at[pl.ds(head * PAL_H + row_base, ROWS_PER_WORKER), :],
        sem)
    pltpu.make_async_copy(
        pal_hbm.at[pl.ds(0, ROWS_PER_WORKER), :],
        ext_v.at[:, pl.ds(0, PAL_W)],
        sem).wait()


def _tc_body(pal_ref, out_ref):
    for g in range(SEQ // PAL_H):
        off = (SEQ - PAL_H) - PAL_H * g
        out_ref[0, pl.ds(PAL_H * g, PAL_H)] = pal_ref[:, off:off + SEQ]


@jax.jit
def _materialize(table_flat):
    sc = functools.partial(
        pl.kernel,
        out_type=jax.ShapeDtypeStruct((NUM_HEADS * PAL_H, PAL_W), jnp.float32),
        mesh=plsc.VectorSubcoreMesh(core_axis_name="c", subcore_axis_name="s"),
        scratch_types=[
            pltpu.VMEM((16384,), jnp.float32),
            pltpu.VMEM((ROWS_PER_WORKER, EXT_PITCH), jnp.float32),
            pltpu.SemaphoreType.DMA,
        ],
        compiler_params=pltpu.CompilerParams(
            needs_layout_passes=False, use_tc_tiling_on_sc=False),
    )(_sc_body)
    pal = sc(table_flat)

    out = pl.pallas_call(
        _tc_body,
        out_shape=jax.ShapeDtypeStruct((NUM_HEADS, SEQ, SEQ), jnp.float32),
        grid=(NUM_HEADS,),
        in_specs=[pl.BlockSpec((PAL_H, PAL_W), lambda h: (h, 0))],
        out_specs=pl.BlockSpec((1, SEQ, SEQ), lambda h: (h, 0, 0)),
    )(pal)
    return out


def kernel(bias_table, seq_len):
    del seq_len
    return _materialize(bias_table.reshape(-1))


# speedup vs baseline: 1.0014x; 1.0014x over previous
import functools

import jax
import jax.numpy as jnp
from jax import lax
from jax.experimental import pallas as pl
from jax.experimental.pallas import tpu as pltpu
from jax.experimental.pallas import tpu_sc as plsc

NUM_HEADS = 16
SEQ = 2048
TBL = 1023
TBL_FLAT = TBL * NUM_HEADS
EXT_PITCH = 4352
LANES = 16
PAL_H = 8
PAL_W = 4088
ROWS_PER_WORKER = PAL_H // 2


def _sc_body(table_hbm, pal_hbm, tbl_v, ext_v, sem):
    head = lax.axis_index("s")
    half = lax.axis_index("c")

    pltpu.sync_copy(table_hbm, tbl_v.at[pl.ds(0, TBL_FLAT)])

    lane = lax.iota(jnp.int32, LANES)
    row_base = half * ROWS_PER_WORKER
    q0 = (PAL_H - 1) - row_base

    def build(it, _):
        base = it * (2 * LANES)
        for u in range(2):
            off = base + u * LANES
            pos = off + lane
            for j in range(ROWS_PER_WORKER):
                r_idx = jnp.clip(pos + (q0 - j) + (-1536), 0, TBL - 1)
                vals = plsc.load_gather(tbl_v, [r_idx * NUM_HEADS + head])
                ext_v[j, pl.ds(off, LANES)] = vals
        return 0

    lax.fori_loop(0, EXT_PITCH // (2 * LANES), build, 0)

    pltpu.async_copy(
        ext_v.at[:, pl.ds(0, PAL_W)],
        pal_hbm.at[pl.ds(head * PAL_H + row_base, ROWS_PER_WORKER), :],
        sem)
    pltpu.make_async_copy(
        pal_hbm.at[pl.ds(0, ROWS_PER_WORKER), :],
        ext_v.at[:, pl.ds(0, PAL_W)],
        sem).wait()


def _tc_body(pal_ref, out_ref):
    for g in range(SEQ // PAL_H):
        off = (SEQ - PAL_H) - PAL_H * g
        out_ref[0, pl.ds(PAL_H * g, PAL_H)] = pal_ref[:, off:off + SEQ]


@jax.jit
def _materialize(table_flat):
    sc = functools.partial(
        pl.kernel,
        out_type=jax.ShapeDtypeStruct((NUM_HEADS * PAL_H, PAL_W), jnp.float32),
        mesh=plsc.VectorSubcoreMesh(core_axis_name="c", subcore_axis_name="s"),
        scratch_types=[
            pltpu.VMEM((16384,), jnp.float32),
            pltpu.VMEM((ROWS_PER_WORKER, EXT_PITCH), jnp.float32),
            pltpu.SemaphoreType.DMA,
        ],
        compiler_params=pltpu.CompilerParams(
            needs_layout_passes=False, use_tc_tiling_on_sc=False),
    )(_sc_body)
    pal = sc(table_flat)

    out = pl.pallas_call(
        _tc_body,
        out_shape=jax.ShapeDtypeStruct((NUM_HEADS, SEQ, SEQ), jnp.float32),
        grid=(NUM_HEADS,),
        in_specs=[pl.BlockSpec((PAL_H, PAL_W), lambda h: (h, 0))],
        out_specs=pl.BlockSpec((1, SEQ, SEQ), lambda h: (h, 0, 0)),
    )(pal)
    return out


def kernel(bias_table, seq_len):
    del seq_len
    return _materialize(bias_table.reshape(-1))
